# Initial kernel scaffold; baseline (speedup 1.0000x reference)
import jax
import jax.numpy as jnp
from jax.experimental import pallas as pl


def kernel(x, i, j, rbf, W_f0, b_f0, W_f2, b_f2, W_lin, b_lin, gamma, beta):
    raise NotImplementedError("write your pallas kernel here")


# speedup vs baseline: 3.6395x; 3.6395x over previous
import functools

import jax
import jax.numpy as jnp
from jax import lax
from jax.experimental import pallas as pl
from jax.experimental.pallas import tpu as pltpu
from jax.experimental.pallas import tpu_sc as plsc

DIM = 128
N_NODES = 10000
N_EDGES = 320000

NC = 2
NS = 16
NW = NC * NS
CHUNK = 128
N_CHUNKS = N_EDGES // CHUNK
CHUNKS_PER_W = N_CHUNKS // NW
CHUNKS_REM = N_CHUNKS % NW
N_PAD = 10240
ROWS_PER_TILE = N_PAD // NS
OUT_COPY = 128


def _lin_body(x_ref, wt_ref, b_ref, o_ref):
    o_ref[...] = (
        jnp.dot(x_ref[...], wt_ref[...], preferred_element_type=jnp.float32)
        + b_ref[...]
    )


def _mlp_body(rbf_ref, wf0t_ref, bf0_ref, wf2t_ref, bf2_ref, o_ref):
    t = jnp.dot(rbf_ref[...], wf0t_ref[...], preferred_element_type=jnp.float32)
    t = jnp.dot(t, wf0t_ref[...], preferred_element_type=jnp.float32) + bf0_ref[...]
    t = t * jax.nn.sigmoid(t)
    o_ref[...] = (
        jnp.dot(t, wf2t_ref[...], preferred_element_type=jnp.float32) + bf2_ref[...]
    )


def _ln_body(x_ref, a0_ref, a1_ref, g_ref, b_ref, o_ref):
    h = x_ref[...] + a0_ref[...] + a1_ref[...]
    mu = jnp.mean(h, axis=-1, keepdims=True)
    var = jnp.mean(jnp.square(h - mu), axis=-1, keepdims=True)
    o_ref[...] = (h - mu) * lax.rsqrt(var + 1e-5) * g_ref[...] + b_ref[...]


def _sc_body(m_hbm, wxh_hbm, i_hbm, j_hbm, out_hbm, jv, iv, rows, mv, agg_sh, sem):
    c = lax.axis_index("c")
    s = lax.axis_index("s")
    wid = s * NC + c

    zeros16 = jnp.zeros((16,), jnp.float32)

    def _zero_row(r, _):
        for q in range(DIM // 16):
            rows[r, pl.ds(q * 16, 16)] = zeros16
        return 0

    lax.fori_loop(0, CHUNK, _zero_row, 0)
    for q in range(ROWS_PER_TILE // OUT_COPY):
        pltpu.sync_copy(
            rows.at[pl.ds(0, OUT_COPY)],
            agg_sh.at[pl.ds(s * ROWS_PER_TILE + q * OUT_COPY, OUT_COPY)],
        )
    plsc.subcore_barrier()

    n_chunks = CHUNKS_PER_W + jnp.where(wid < CHUNKS_REM, 1, 0)

    def _chunk(g, _):
        off = (g * NW + wid) * CHUNK
        pltpu.sync_copy(j_hbm.at[pl.ds(off, CHUNK)], jv)
        pltpu.sync_copy(i_hbm.at[pl.ds(off, CHUNK)], iv)
        gather = pltpu.async_copy(wxh_hbm.at[jv], rows, sem)
        pltpu.sync_copy(m_hbm.at[pl.ds(off, CHUNK)], mv)
        gather.wait()

        def _mul_row(r, _):
            for q in range(DIM // 16):
                sl = pl.ds(q * 16, 16)
                rows[r, sl] = rows[r, sl] * mv[r, sl]
            return 0

        lax.fori_loop(0, CHUNK, _mul_row, 0)
        pltpu.sync_copy(rows, agg_sh.at[iv], add=True)
        return 0

    lax.fori_loop(0, n_chunks, _chunk, 0)
    plsc.subcore_barrier()

    for q in range(ROWS_PER_TILE // OUT_COPY):
        r0 = s * ROWS_PER_TILE + q * OUT_COPY
        pltpu.sync_copy(
            agg_sh.at[pl.ds(r0, OUT_COPY)], out_hbm.at[c, pl.ds(r0, OUT_COPY)]
        )


_sc_call = functools.partial(
    pl.kernel,
    out_type=jax.ShapeDtypeStruct((NC, N_PAD, DIM), jnp.float32),
    mesh=plsc.VectorSubcoreMesh(core_axis_name="c", subcore_axis_name="s"),
    scratch_types=[
        pltpu.VMEM((CHUNK,), jnp.int32),
        pltpu.VMEM((CHUNK,), jnp.int32),
        pltpu.VMEM((CHUNK, DIM), jnp.float32),
        pltpu.VMEM((CHUNK, DIM), jnp.float32),
        pltpu.VMEM_SHARED((N_PAD, DIM), jnp.float32),
        pltpu.SemaphoreType.DMA,
    ],
)(_sc_body)


def kernel(x, i, j, rbf, W_f0, b_f0, W_f2, b_f2, W_lin, b_lin, gamma, beta):
    i = i.astype(jnp.int32)
    j = j.astype(jnp.int32)

    nb = 10
    rows_b = N_NODES // nb
    wxh = pl.pallas_call(
        _lin_body,
        grid=(nb,),
        in_specs=[
            pl.BlockSpec((rows_b, DIM), lambda b: (b, 0)),
            pl.BlockSpec((DIM, DIM), lambda b: (0, 0)),
            pl.BlockSpec((1, DIM), lambda b: (0, 0)),
        ],
        out_specs=pl.BlockSpec((rows_b, DIM), lambda b: (b, 0)),
        out_shape=jax.ShapeDtypeStruct((N_NODES, DIM), jnp.float32),
    )(x, W_lin.T, b_lin.reshape(1, DIM))

    eb = 4000
    m = pl.pallas_call(
        _mlp_body,
        grid=(N_EDGES // eb,),
        in_specs=[
            pl.BlockSpec((eb, DIM), lambda b: (b, 0)),
            pl.BlockSpec((DIM, DIM), lambda b: (0, 0)),
            pl.BlockSpec((1, DIM), lambda b: (0, 0)),
            pl.BlockSpec((DIM, DIM), lambda b: (0, 0)),
            pl.BlockSpec((1, DIM), lambda b: (0, 0)),
        ],
        out_specs=pl.BlockSpec((eb, DIM), lambda b: (b, 0)),
        out_shape=jax.ShapeDtypeStruct((N_EDGES, DIM), jnp.float32),
    )(rbf, W_f0.T, b_f0.reshape(1, DIM), W_f2.T, b_f2.reshape(1, DIM))

    agg2 = _sc_call(m, wxh, i, j)[:, :N_NODES, :]

    out = pl.pallas_call(
        _ln_body,
        grid=(nb,),
        in_specs=[
            pl.BlockSpec((rows_b, DIM), lambda b: (b, 0)),
            pl.BlockSpec((rows_b, DIM), lambda b: (b, 0)),
            pl.BlockSpec((rows_b, DIM), lambda b: (b, 0)),
            pl.BlockSpec((1, DIM), lambda b: (0, 0)),
            pl.BlockSpec((1, DIM), lambda b: (0, 0)),
        ],
        out_specs=pl.BlockSpec((rows_b, DIM), lambda b: (b, 0)),
        out_shape=jax.ShapeDtypeStruct((N_NODES, DIM), jnp.float32),
    )(x, agg2[0], agg2[1], gamma.reshape(1, DIM), beta.reshape(1, DIM))
    return out


# speedup vs baseline: 4.2494x; 1.1676x over previous
import functools

import jax
import jax.numpy as jnp
from jax import lax
from jax.experimental import pallas as pl
from jax.experimental.pallas import tpu as pltpu
from jax.experimental.pallas import tpu_sc as plsc

DIM = 128
N_NODES = 10000
N_EDGES = 320000

NC = 2
NS = 16
NW = NC * NS
CHUNK = 128
N_CHUNKS = N_EDGES // CHUNK
CHUNKS_PER_W = N_CHUNKS // NW
CHUNKS_REM = N_CHUNKS % NW
N_PAD = 10112
ROWS_PER_TILE = N_PAD // NS
OUT_COPY = 128
OUT_TAIL = ROWS_PER_TILE - 4 * OUT_COPY
HALF = CHUNKS_PER_W // 2


def _lin_body(x_ref, wt_ref, b_ref, o_ref):
    o_ref[...] = (
        jnp.dot(x_ref[...], wt_ref[...], preferred_element_type=jnp.float32)
        + b_ref[...]
    )


def _mlp_body(rbf_ref, wf0t_ref, bf0_ref, wf2t_ref, bf2_ref, o_ref):
    t = jnp.dot(rbf_ref[...], wf0t_ref[...], preferred_element_type=jnp.float32)
    t = jnp.dot(t, wf0t_ref[...], preferred_element_type=jnp.float32) + bf0_ref[...]
    t = t * jax.nn.sigmoid(t)
    o_ref[...] = (
        jnp.dot(t, wf2t_ref[...], preferred_element_type=jnp.float32) + bf2_ref[...]
    )


def _ln_body(x_ref, a0_ref, a1_ref, g_ref, b_ref, o_ref):
    h = x_ref[...] + a0_ref[...] + a1_ref[...]
    mu = jnp.mean(h, axis=-1, keepdims=True)
    var = jnp.mean(jnp.square(h - mu), axis=-1, keepdims=True)
    o_ref[...] = (h - mu) * lax.rsqrt(var + 1e-5) * g_ref[...] + b_ref[...]


def _sc_body(m_hbm, wxh_hbm, i_hbm, j_hbm, out_hbm,
             jv0, iv0, rows0, jv1, iv1, rows1, mv,
             agg_sh, si0, sg0, ss0, si1, sg1, ss1):
    c = lax.axis_index("c")
    s = lax.axis_index("s")
    wid = s * NC + c

    zeros16 = jnp.zeros((16,), jnp.float32)

    def _zero_row(r, _):
        for q in range(DIM // 16):
            rows0[r, pl.ds(q * 16, 16)] = zeros16
        return 0

    lax.fori_loop(0, CHUNK, _zero_row, 0)
    for q in range(4):
        pltpu.sync_copy(
            rows0.at[pl.ds(0, OUT_COPY)],
            agg_sh.at[pl.ds(s * ROWS_PER_TILE + q * OUT_COPY, OUT_COPY)],
        )
    pltpu.sync_copy(
        rows0.at[pl.ds(0, OUT_TAIL)],
        agg_sh.at[pl.ds(s * ROWS_PER_TILE + 4 * OUT_COPY, OUT_TAIL)],
    )
    plsc.subcore_barrier()

    bufs = ((jv0, iv0, rows0, si0, sg0, ss0),
            (jv1, iv1, rows1, si1, sg1, ss1))

    def _off(g):
        return (g * NW + wid) * CHUNK

    def issue_idx(g, b):
        jv, iv, _, si, _, _ = bufs[b]
        pltpu.async_copy(j_hbm.at[pl.ds(_off(g), CHUNK)], jv, si)
        pltpu.async_copy(i_hbm.at[pl.ds(_off(g), CHUNK)], iv, si)

    def wait_idx(b):
        jv, iv, _, si, _, _ = bufs[b]
        pltpu.make_async_copy(j_hbm.at[pl.ds(0, CHUNK)], jv, si).wait()
        pltpu.make_async_copy(i_hbm.at[pl.ds(0, CHUNK)], iv, si).wait()

    def issue_g(b):
        jv, _, rows, _, sg, _ = bufs[b]
        pltpu.async_copy(wxh_hbm.at[jv], rows, sg)

    def wait_g(b):
        jv, _, rows, _, sg, _ = bufs[b]
        pltpu.make_async_copy(wxh_hbm.at[jv], rows, sg).wait()

    def copy_m(g):
        pltpu.sync_copy(m_hbm.at[pl.ds(_off(g), CHUNK)], mv)

    def mul(b):
        _, _, rows, _, _, _ = bufs[b]

        def _mul_row(r, _):
            for q in range(DIM // 16):
                sl = pl.ds(q * 16, 16)
                rows[r, sl] = rows[r, sl] * mv[r, sl]
            return 0

        lax.fori_loop(0, CHUNK, _mul_row, 0)

    def scatter_start(b):
        _, iv, rows, _, _, ss = bufs[b]
        pltpu.async_copy(rows, agg_sh.at[iv], ss, add=True)

    def scatter_wait(b):
        _, iv, rows, _, _, ss = bufs[b]
        pltpu.make_async_copy(rows, agg_sh.at[iv], ss).wait()

    issue_idx(0, 0)
    wait_idx(0)
    issue_g(0)
    issue_idx(1, 1)

    def _steady(t, _):
        wait_idx(1)
        issue_g(1)
        copy_m(2 * t)
        wait_g(0)
        mul(0)
        scatter_start(0)
        copy_m(2 * t + 1)
        scatter_wait(0)
        issue_idx(2 * t + 2, 0)
        wait_idx(0)
        issue_g(0)
        wait_g(1)
        mul(1)
        scatter_start(1)
        scatter_wait(1)
        issue_idx(2 * t + 3, 1)
        return 0

    lax.fori_loop(0, HALF - 1, _steady, 0)

    wait_idx(1)
    issue_g(1)
    copy_m(CHUNKS_PER_W - 2)
    wait_g(0)
    mul(0)
    scatter_start(0)
    copy_m(CHUNKS_PER_W - 1)
    scatter_wait(0)
    wait_g(1)
    mul(1)
    scatter_start(1)
    scatter_wait(1)

    @pl.when(wid < CHUNKS_REM)
    def _():
        off = (CHUNKS_PER_W * NW + wid) * CHUNK
        pltpu.sync_copy(j_hbm.at[pl.ds(off, CHUNK)], jv0)
        pltpu.sync_copy(i_hbm.at[pl.ds(off, CHUNK)], iv0)
        pltpu.async_copy(wxh_hbm.at[jv0], rows0, sg0).wait()
        pltpu.sync_copy(m_hbm.at[pl.ds(off, CHUNK)], mv)

        def _mul_row(r, _):
            for q in range(DIM // 16):
                sl = pl.ds(q * 16, 16)
                rows0[r, sl] = rows0[r, sl] * mv[r, sl]
            return 0

        lax.fori_loop(0, CHUNK, _mul_row, 0)
        pltpu.sync_copy(rows0, agg_sh.at[iv0], add=True)

    plsc.subcore_barrier()
    for q in range(4):
        r0 = s * ROWS_PER_TILE + q * OUT_COPY
        pltpu.sync_copy(
            agg_sh.at[pl.ds(r0, OUT_COPY)], out_hbm.at[c, pl.ds(r0, OUT_COPY)]
        )
    r0 = s * ROWS_PER_TILE + 4 * OUT_COPY
    pltpu.sync_copy(
        agg_sh.at[pl.ds(r0, OUT_TAIL)], out_hbm.at[c, pl.ds(r0, OUT_TAIL)]
    )


_sc_call = functools.partial(
    pl.kernel,
    out_type=jax.ShapeDtypeStruct((NC, N_PAD, DIM), jnp.float32),
    mesh=plsc.VectorSubcoreMesh(core_axis_name="c", subcore_axis_name="s"),
    scratch_types=[
        pltpu.VMEM((CHUNK,), jnp.int32),
        pltpu.VMEM((CHUNK,), jnp.int32),
        pltpu.VMEM((CHUNK, DIM), jnp.float32),
        pltpu.VMEM((CHUNK,), jnp.int32),
        pltpu.VMEM((CHUNK,), jnp.int32),
        pltpu.VMEM((CHUNK, DIM), jnp.float32),
        pltpu.VMEM((CHUNK, DIM), jnp.float32),
        pltpu.VMEM_SHARED((N_PAD, DIM), jnp.float32),
        pltpu.SemaphoreType.DMA,
        pltpu.SemaphoreType.DMA,
        pltpu.SemaphoreType.DMA,
        pltpu.SemaphoreType.DMA,
        pltpu.SemaphoreType.DMA,
        pltpu.SemaphoreType.DMA,
    ],
)(_sc_body)


def kernel(x, i, j, rbf, W_f0, b_f0, W_f2, b_f2, W_lin, b_lin, gamma, beta):
    i = i.astype(jnp.int32)
    j = j.astype(jnp.int32)

    nb = 10
    rows_b = N_NODES // nb
    wxh = pl.pallas_call(
        _lin_body,
        grid=(nb,),
        in_specs=[
            pl.BlockSpec((rows_b, DIM), lambda b: (b, 0)),
            pl.BlockSpec((DIM, DIM), lambda b: (0, 0)),
            pl.BlockSpec((1, DIM), lambda b: (0, 0)),
        ],
        out_specs=pl.BlockSpec((rows_b, DIM), lambda b: (b, 0)),
        out_shape=jax.ShapeDtypeStruct((N_NODES, DIM), jnp.float32),
    )(x, W_lin.T, b_lin.reshape(1, DIM))

    eb = 4000
    m = pl.pallas_call(
        _mlp_body,
        grid=(N_EDGES // eb,),
        in_specs=[
            pl.BlockSpec((eb, DIM), lambda b: (b, 0)),
            pl.BlockSpec((DIM, DIM), lambda b: (0, 0)),
            pl.BlockSpec((1, DIM), lambda b: (0, 0)),
            pl.BlockSpec((DIM, DIM), lambda b: (0, 0)),
            pl.BlockSpec((1, DIM), lambda b: (0, 0)),
        ],
        out_specs=pl.BlockSpec((eb, DIM), lambda b: (b, 0)),
        out_shape=jax.ShapeDtypeStruct((N_EDGES, DIM), jnp.float32),
    )(rbf, W_f0.T, b_f0.reshape(1, DIM), W_f2.T, b_f2.reshape(1, DIM))

    agg2 = _sc_call(m, wxh, i, j)[:, :N_NODES, :]

    out = pl.pallas_call(
        _ln_body,
        grid=(nb,),
        in_specs=[
            pl.BlockSpec((rows_b, DIM), lambda b: (b, 0)),
            pl.BlockSpec((rows_b, DIM), lambda b: (b, 0)),
            pl.BlockSpec((rows_b, DIM), lambda b: (b, 0)),
            pl.BlockSpec((1, DIM), lambda b: (0, 0)),
            pl.BlockSpec((1, DIM), lambda b: (0, 0)),
        ],
        out_specs=pl.BlockSpec((rows_b, DIM), lambda b: (b, 0)),
        out_shape=jax.ShapeDtypeStruct((N_NODES, DIM), jnp.float32),
    )(x, agg2[0], agg2[1], gamma.reshape(1, DIM), beta.reshape(1, DIM))
    return out
